# Initial kernel scaffold; baseline (speedup 1.0000x reference)
import jax
import jax.numpy as jnp
from jax.experimental import pallas as pl


def kernel(x, qr, cos, sin, mask, Wq, Wk, Wp, g):
    raise NotImplementedError("write your pallas kernel here")


# speedup vs baseline: 1.0032x; 1.0032x over previous
import functools

import jax
import jax.numpy as jnp
import numpy as np
from jax.experimental import pallas as pl
from jax.experimental.pallas import tpu as pltpu

_N_HEADS = 16
_HEAD_DIM = 128
_ROPE_DIM = 64
_S = 2048
_DIM = 3072
_QLORA = 1536
_EPS = 1e-6
_SCALE = _HEAD_DIM ** (-0.5)
_MINVAL = float(jnp.finfo(jnp.bfloat16).min)
_WSCALE = float(jnp.bfloat16(_N_HEADS ** (-0.5) * _SCALE))

_PERM = np.concatenate([np.arange(0, 64, 2), np.arange(1, 64, 2), np.arange(64, 128)])


def _kprep_body(x_ref, wk_ref, g_ref, cos_ref, sin_ref, k_out):
    k32 = jax.lax.dot_general(x_ref[...], wk_ref[...],
                              dimension_numbers=(((1,), (1,)), ((), ())),
                              preferred_element_type=jnp.float32)
    ms = jnp.mean(k32 * k32, axis=-1, keepdims=True)
    kr = k32 * jax.lax.rsqrt(ms + _EPS) * g_ref[...]
    kp = kr[:, :_ROPE_DIM]
    rot = jnp.concatenate([-kp[:, 32:], kp[:, :32]], axis=1)
    kpe = kp * cos_ref[...] + rot * sin_ref[...]
    k_out[...] = jnp.concatenate([kpe, kr[:, _ROPE_DIM:]], axis=1)


def _score_body(qr_ref, wq_ref, x_ref, wp_ref, cos_ref, sin_ref, k_ref, mask_ref,
                score_out):
    q32 = jax.lax.dot_general(qr_ref[...], wq_ref[...],
                              dimension_numbers=(((1,), (1,)), ((), ())),
                              preferred_element_type=jnp.float32)
    pw = jax.lax.dot_general(x_ref[...], wp_ref[...],
                             dimension_numbers=(((1,), (1,)), ((), ())),
                             preferred_element_type=jnp.float32)
    w = pw * jnp.float32(_WSCALE)
    cos = cos_ref[...]
    sin = sin_ref[...]
    kb16 = k_ref[...].astype(jnp.bfloat16)
    acc = mask_ref[...]
    for h in range(_N_HEADS):
        base = h * _HEAD_DIM
        qpe = q32[:, base:base + _ROPE_DIM]
        rot = jnp.concatenate([-qpe[:, 32:], qpe[:, :32]], axis=1)
        qpe = qpe * cos + rot * sin
        qh = jnp.concatenate([qpe, q32[:, base + _ROPE_DIM:base + _HEAD_DIM]], axis=1)
        sh = jax.lax.dot_general(qh.astype(jnp.bfloat16), kb16,
                                 dimension_numbers=(((1,), (1,)), ((), ())),
                                 preferred_element_type=jnp.float32)
        acc = acc + w[:, h:h + 1] * jnp.maximum(sh, 0.0)
    score_out[...] = acc


def _compute_scores(x2, qr2, cos2, sin2, mask2, wq_p, wk_p, wp, g_p):
    KB = 512
    k_all = pl.pallas_call(
        _kprep_body,
        grid=(_S // KB,),
        in_specs=[
            pl.BlockSpec((KB, _DIM), lambda i: (i, 0)),
            pl.BlockSpec((_HEAD_DIM, _DIM), lambda i: (0, 0)),
            pl.BlockSpec((1, _HEAD_DIM), lambda i: (0, 0)),
            pl.BlockSpec((KB, _ROPE_DIM), lambda i: (i, 0)),
            pl.BlockSpec((KB, _ROPE_DIM), lambda i: (i, 0)),
        ],
        out_specs=pl.BlockSpec((KB, _HEAD_DIM), lambda i: (i, 0)),
        out_shape=jax.ShapeDtypeStruct((_S, _HEAD_DIM), jnp.float32),
    )(x2, wk_p, g_p, cos2, sin2)

    RB = 256
    score = pl.pallas_call(
        _score_body,
        grid=(_S // RB,),
        in_specs=[
            pl.BlockSpec((RB, _QLORA), lambda i: (i, 0)),
            pl.BlockSpec((_N_HEADS * _HEAD_DIM, _QLORA), lambda i: (0, 0)),
            pl.BlockSpec((RB, _DIM), lambda i: (i, 0)),
            pl.BlockSpec((_N_HEADS, _DIM), lambda i: (0, 0)),
            pl.BlockSpec((RB, _ROPE_DIM), lambda i: (i, 0)),
            pl.BlockSpec((RB, _ROPE_DIM), lambda i: (i, 0)),
            pl.BlockSpec((_S, _HEAD_DIM), lambda i: (0, 0)),
            pl.BlockSpec((RB, _S), lambda i: (i, 0)),
        ],
        out_specs=pl.BlockSpec((RB, _S), lambda i: (i, 0)),
        out_shape=jax.ShapeDtypeStruct((_S, _S), jnp.float32),
    )(qr2, wq_p, x2, wp, cos2, sin2, k_all, mask2)
    return score


def kernel(x, qr, cos, sin, mask, Wq, Wk, Wp, g):
    B, S, _ = x.shape
    x2 = x.reshape(S, _DIM)
    qr2 = qr.reshape(S, _QLORA)
    cos2 = cos.reshape(S, _ROPE_DIM)
    sin2 = sin.reshape(S, _ROPE_DIM)
    mask2 = mask.reshape(S, S)
    perm = jnp.asarray(_PERM)
    wq_p = Wq.reshape(_N_HEADS, _HEAD_DIM, _QLORA)[:, perm, :].reshape(
        _N_HEADS * _HEAD_DIM, _QLORA)
    wk_p = Wk[perm, :]
    g_p = g[perm].reshape(1, _HEAD_DIM)

    score = _compute_scores(x2, qr2, cos2, sin2, mask2, wq_p, wk_p, Wp, g_p)

    index_score = score.reshape(B, 1, S, S)
    topk = min(2048, S)
    _, topk_indices = jax.lax.top_k(index_score, topk)
    future_mask = jnp.arange(S)[None, None, :, None] < jnp.arange(topk)[None, None, None, :]
    topk_indices = jnp.where(future_mask, S, topk_indices)
    index_mask = jnp.full((B, 1, S + 1, S + 1), _MINVAL, dtype=jnp.float32)
    bi = jnp.arange(B)[:, None, None, None]
    hi = jnp.zeros((1, 1, 1, 1), dtype=jnp.int32)
    ri = jnp.arange(S)[None, None, :, None]
    index_mask = index_mask.at[bi, hi, ri, topk_indices].set(0.0)
    index_mask = index_mask[:, :, :S, :S]
    out_score = jnp.clip(index_score + index_mask, _MINVAL, None)
    return topk_indices, out_score, index_mask


# speedup vs baseline: 5.8649x; 5.8461x over previous
import functools

import jax
import jax.numpy as jnp
import numpy as np
from jax.experimental import pallas as pl
from jax.experimental.pallas import tpu as pltpu
from jax.experimental.pallas import tpu_sc as plsc

_N_HEADS = 16
_HEAD_DIM = 128
_ROPE_DIM = 64
_S = 2048
_DIM = 3072
_QLORA = 1536
_EPS = 1e-6
_SCALE = _HEAD_DIM ** (-0.5)
_MINVAL = float(jnp.finfo(jnp.bfloat16).min)
_WSCALE = 0.0220947265625

_PERM = np.concatenate([np.arange(0, 64, 2), np.arange(1, 64, 2), np.arange(64, 128)])


def _kprep_body(x_ref, wk_ref, g_ref, cos_ref, sin_ref, k_out):
    k32 = jax.lax.dot_general(x_ref[...], wk_ref[...],
                              dimension_numbers=(((1,), (1,)), ((), ())),
                              preferred_element_type=jnp.float32)
    ms = jnp.mean(k32 * k32, axis=-1, keepdims=True)
    kr = k32 * jax.lax.rsqrt(ms + _EPS) * g_ref[...]
    kp = kr[:, :_ROPE_DIM]
    rot = jnp.concatenate([-kp[:, 32:], kp[:, :32]], axis=1)
    kpe = kp * cos_ref[...] + rot * sin_ref[...]
    k_out[...] = jnp.concatenate([kpe, kr[:, _ROPE_DIM:]], axis=1)


def _score_body(qr_ref, wq_ref, x_ref, wp_ref, cos_ref, sin_ref, k_ref, mask_ref,
                score_out, key_out):
    q32 = jax.lax.dot_general(qr_ref[...], wq_ref[...],
                              dimension_numbers=(((1,), (1,)), ((), ())),
                              preferred_element_type=jnp.float32)
    pw = jax.lax.dot_general(x_ref[...], wp_ref[...],
                             dimension_numbers=(((1,), (1,)), ((), ())),
                             preferred_element_type=jnp.float32)
    w = pw * jnp.float32(_WSCALE)
    cos = cos_ref[...]
    sin = sin_ref[...]
    kb16 = k_ref[...].astype(jnp.bfloat16)
    acc = mask_ref[...]
    for h in range(_N_HEADS):
        base = h * _HEAD_DIM
        qpe = q32[:, base:base + _ROPE_DIM]
        rot = jnp.concatenate([-qpe[:, 32:], qpe[:, :32]], axis=1)
        qpe = qpe * cos + rot * sin
        qh = jnp.concatenate([qpe, q32[:, base + _ROPE_DIM:base + _HEAD_DIM]], axis=1)
        sh = jax.lax.dot_general(qh.astype(jnp.bfloat16), kb16,
                                 dimension_numbers=(((1,), (1,)), ((), ())),
                                 preferred_element_type=jnp.float32)
        acc = acc + w[:, h:h + 1] * jnp.maximum(sh, 0.0)
    score_out[...] = acc
    b = jax.lax.bitcast_convert_type(acc, jnp.int32)
    key_out[...] = jnp.where(b < 0, b ^ np.int32(-2 ** 31), ~b)


def _compute_scores(x2, qr2, cos2, sin2, mask2, wq_p, wk_p, wp, g_p):
    KB = 512
    k_all = pl.pallas_call(
        _kprep_body,
        grid=(_S // KB,),
        in_specs=[
            pl.BlockSpec((KB, _DIM), lambda i: (i, 0)),
            pl.BlockSpec((_HEAD_DIM, _DIM), lambda i: (0, 0)),
            pl.BlockSpec((1, _HEAD_DIM), lambda i: (0, 0)),
            pl.BlockSpec((KB, _ROPE_DIM), lambda i: (i, 0)),
            pl.BlockSpec((KB, _ROPE_DIM), lambda i: (i, 0)),
        ],
        out_specs=pl.BlockSpec((KB, _HEAD_DIM), lambda i: (i, 0)),
        out_shape=jax.ShapeDtypeStruct((_S, _HEAD_DIM), jnp.float32),
    )(x2, wk_p, g_p, cos2, sin2)

    RB = 256
    score, key = pl.pallas_call(
        _score_body,
        grid=(_S // RB,),
        in_specs=[
            pl.BlockSpec((RB, _QLORA), lambda i: (i, 0)),
            pl.BlockSpec((_N_HEADS * _HEAD_DIM, _QLORA), lambda i: (0, 0)),
            pl.BlockSpec((RB, _DIM), lambda i: (i, 0)),
            pl.BlockSpec((_N_HEADS, _DIM), lambda i: (0, 0)),
            pl.BlockSpec((RB, _ROPE_DIM), lambda i: (i, 0)),
            pl.BlockSpec((RB, _ROPE_DIM), lambda i: (i, 0)),
            pl.BlockSpec((_S, _HEAD_DIM), lambda i: (0, 0)),
            pl.BlockSpec((RB, _S), lambda i: (i, 0)),
        ],
        out_specs=(pl.BlockSpec((RB, _S), lambda i: (i, 0)),
                   pl.BlockSpec((RB, _S), lambda i: (i, 0))),
        out_shape=(jax.ShapeDtypeStruct((_S, _S), jnp.float32),
                   jax.ShapeDtypeStruct((_S, _S), jnp.int32)),
    )(qr2, wq_p, x2, wp, cos2, sin2, k_all, mask2)
    return score, key


_NW = 32
_RPW = _S // _NW
_TOP = np.int32(-(2 ** 31))


def _sc_sort_rows(score2d, key2d):
    mesh = plsc.VectorSubcoreMesh(core_axis_name="c", subcore_axis_name="s")

    @functools.partial(
        pl.kernel,
        mesh=mesh,
        out_type=(jax.ShapeDtypeStruct((_S, _S), jnp.int32),
                  jax.ShapeDtypeStruct((_S, _S), jnp.float32),
                  jax.ShapeDtypeStruct((_S, _S), jnp.float32)),
        scratch_types=[
            pltpu.VMEM((_S,), jnp.float32),
            pltpu.VMEM((_S,), jnp.int32),
            pltpu.VMEM((_S,), jnp.int32),
            pltpu.VMEM((_S,), jnp.int32),
            pltpu.VMEM((_S,), jnp.int32),
            pltpu.VMEM((512,), jnp.int32),
            pltpu.VMEM((512,), jnp.int32),
            pltpu.VMEM((_S,), jnp.int32),
            pltpu.VMEM((_S,), jnp.float32),
            pltpu.VMEM((_S,), jnp.float32),
            pltpu.VMEM((_S,), jnp.int32),
        ],
        compiler_params=pltpu.CompilerParams(needs_layout_passes=False),
    )
    def sorter(score_hbm, key_hbm, idx_hbm, mask_hbm, sco_hbm,
               srow, key_a, key_b, idx_a, idx_b, hist, hoff, kept,
               mrow, orow, irow):
        lane = jax.lax.iota(jnp.int32, 16)
        gidx0 = lane * 128
        ones = jnp.ones((16,), jnp.int32)
        zeros = jnp.zeros((16,), jnp.int32)
        minv = jnp.full((16,), _MINVAL, jnp.float32)
        zf = jnp.zeros((16,), jnp.float32)
        wid = jax.lax.axis_index("s") * 2 + jax.lax.axis_index("c")
        base_row = wid * _RPW

        def radix_pass(src_key, src_idx, dst_key, dst_idx, p):
            shift = 5 * p
            nb = 32 if p < 6 else 4

            def clr(i, c):
                hist[pl.ds(i * 16, 16)] = zeros
                return c
            jax.lax.fori_loop(0, nb, clr, 0)

            def histf(i, c):
                kv = plsc.load_gather(src_key, [gidx0 + i])
                dg = jax.lax.shift_right_logical(kv, shift) & 31
                if p == 6:
                    dg = (dg & 3) ^ 2
                plsc.addupdate_scatter(hist, [dg * 16 + lane], ones)
                return c
            jax.lax.fori_loop(0, 128, histf, 0)

            def scanf(i, carry):
                v = hist[pl.ds(i * 16, 16)]
                cs = plsc.cumsum(v)
                hoff[pl.ds(i * 16, 16)] = cs - v + carry
                return carry + jnp.sum(v)
            jax.lax.fori_loop(0, nb, scanf, jnp.int32(0))

            def permf(i, c):
                gi = gidx0 + i
                kv = plsc.load_gather(src_key, [gi])
                iv = gi if src_idx is None else plsc.load_gather(src_idx, [gi])
                dg = jax.lax.shift_right_logical(kv, shift) & 31
                if p == 6:
                    dg = (dg & 3) ^ 2
                bins = dg * 16 + lane
                off = plsc.load_gather(hoff, [bins])
                plsc.store_scatter(dst_key, [off], kv)
                plsc.store_scatter(dst_idx, [off], iv)
                plsc.addupdate_scatter(hoff, [bins], ones)
                return c
            jax.lax.fori_loop(0, 128, permf, 0)

        def do_row(j, c):
            r = base_row + j
            pltpu.sync_copy(score_hbm.at[r], srow)
            pltpu.sync_copy(key_hbm.at[r], key_a)

            radix_pass(key_a, None, key_b, idx_b, 0)
            radix_pass(key_b, idx_b, key_a, idx_a, 1)
            radix_pass(key_a, idx_a, key_b, idx_b, 2)
            radix_pass(key_b, idx_b, key_a, idx_a, 3)
            radix_pass(key_a, idx_a, key_b, idx_b, 4)
            radix_pass(key_b, idx_b, key_a, idx_a, 5)
            radix_pass(key_a, idx_a, key_b, idx_b, 6)

            def tailf(i, cc):
                pos = lane + i * 16
                v = idx_b[pl.ds(i * 16, 16)]
                irow[pl.ds(i * 16, 16)] = jnp.where(pos <= r, v, jnp.int32(_S))
                kept[pl.ds(i * 16, 16)] = zeros
                return cc
            jax.lax.fori_loop(0, 128, tailf, 0)
            pltpu.sync_copy(irow, idx_hbm.at[r])

            def ksetf(i, cc):
                t = lane + i * 16
                idv = idx_b[pl.ds(i * 16, 16)]
                plsc.store_scatter(kept, [idv], ones, mask=t <= r)
                return cc
            jax.lax.fori_loop(0, (r >> 4) + 1, ksetf, 0)

            def emitf(i, cc):
                kv = kept[pl.ds(i * 16, 16)]
                sv = srow[pl.ds(i * 16, 16)]
                keep = kv > 0
                mrow[pl.ds(i * 16, 16)] = jnp.where(keep, zf, minv)
                orow[pl.ds(i * 16, 16)] = jnp.where(keep, sv, minv)
                return cc
            jax.lax.fori_loop(0, 128, emitf, 0)
            pltpu.sync_copy(mrow, mask_hbm.at[r])
            pltpu.sync_copy(orow, sco_hbm.at[r])
            return c

        jax.lax.fori_loop(0, _RPW, do_row, 0)

    return sorter(score2d, key2d)


def kernel(x, qr, cos, sin, mask, Wq, Wk, Wp, g):
    B, S, _ = x.shape
    x2 = x.reshape(S, _DIM)
    qr2 = qr.reshape(S, _QLORA)
    cos2 = cos.reshape(S, _ROPE_DIM)
    sin2 = sin.reshape(S, _ROPE_DIM)
    mask2 = mask.reshape(S, S)
    perm = jnp.asarray(_PERM)
    wq_p = Wq.reshape(_N_HEADS, _HEAD_DIM, _QLORA)[:, perm, :].reshape(
        _N_HEADS * _HEAD_DIM, _QLORA)
    wk_p = Wk[perm, :]
    g_p = g[perm].reshape(1, _HEAD_DIM)

    score, key = _compute_scores(x2, qr2, cos2, sin2, mask2, wq_p, wk_p, Wp, g_p)

    idx2d, mask2d, sco2d = _sc_sort_rows(score, key)
    return (idx2d.reshape(B, 1, S, S),
            sco2d.reshape(B, 1, S, S),
            mask2d.reshape(B, 1, S, S))
